# baseline (speedup 1.0000x reference)
import functools

import jax
import jax.numpy as jnp
from jax import lax
from jax.experimental import pallas as pl
from jax.experimental.pallas import tpu as pltpu
from jax.experimental.pallas import tpu_sc as plsc

N_TASKS = 80
TASK_DIM = 96
BATCH = 16384
OBS_DIM = 512
OUT_DIM = OBS_DIM + TASK_DIM
_TPAD = 96
_L = 16


def _renorm_body(w_ref, out_ref):
    w = w_ref[...]
    ss = jnp.sum(w * w, axis=1, keepdims=True)
    scale = jnp.where(ss > 1.0, lax.rsqrt(ss), 1.0)
    wt = (w * scale).T
    out_ref[...] = jnp.concatenate(
        [wt, jnp.zeros((TASK_DIM, _TPAD - N_TASKS), jnp.float32)], axis=1
    )


def _renorm_table_t(w):
    return pl.pallas_call(
        _renorm_body,
        out_shape=jax.ShapeDtypeStruct((TASK_DIM, _TPAD), jnp.float32),
    )(w)


_info = plsc.get_sparse_core_info()
_NC = _info.num_cores
_NS = _info.num_subcores
_NW = _NC * _NS
_B_PER_W = BATCH // _NW


@functools.partial(
    pl.kernel,
    mesh=plsc.VectorSubcoreMesh(core_axis_name="c", subcore_axis_name="s"),
    out_type=jax.ShapeDtypeStruct((TASK_DIM, BATCH), jnp.float32),
    compiler_params=pltpu.CompilerParams(needs_layout_passes=False),
    scratch_types=[
        pltpu.VMEM((_B_PER_W,), jnp.int32),
        pltpu.VMEM((TASK_DIM, _TPAD), jnp.float32),
        pltpu.VMEM((TASK_DIM, _B_PER_W), jnp.float32),
    ],
)
def _sc_gather_t(task_hbm, table_hbm, embt_hbm, idx_v, table_v, embt_v):
    wid = lax.axis_index("s") * _NC + lax.axis_index("c")
    base = wid * _B_PER_W
    pltpu.sync_copy(task_hbm.at[pl.ds(base, _B_PER_W)], idx_v)
    pltpu.sync_copy(table_hbm, table_v)

    def group(g, carry):
        idx16 = idx_v[pl.ds(g * _L, _L)]
        for c in range(TASK_DIM):
            row = jnp.full((_L,), c, jnp.int32)
            embt_v[c, pl.ds(g * _L, _L)] = plsc.load_gather(table_v, [row, idx16])
        return carry

    lax.fori_loop(0, _B_PER_W // _L, group, 0)
    pltpu.sync_copy(embt_v, embt_hbm.at[:, pl.ds(base, _B_PER_W)])


_TB = 4096
_EB = 16
_TTB = 16384


def _obs_body(obs_ref, out_ref):
    out_ref[...] = obs_ref[...].T


def _tail_body(_, embt_ref, out_ref):
    out_ref[...] = embt_ref[...]


def _assemble(obs, embt):
    out_t0 = pl.pallas_call(
        _obs_body,
        grid=(BATCH // _TB,),
        in_specs=[pl.BlockSpec((_TB, OBS_DIM), lambda i: (i, 0))],
        out_specs=pl.BlockSpec((OBS_DIM, _TB), lambda i: (0, i)),
        out_shape=jax.ShapeDtypeStruct((OUT_DIM, BATCH), jnp.float32),
    )(obs)
    out_t = pl.pallas_call(
        _tail_body,
        grid=(BATCH // _TTB, TASK_DIM // _EB),
        in_specs=[
            pl.BlockSpec(memory_space=pl.ANY),
            pl.BlockSpec((_EB, _TTB), lambda i, r: (r, i)),
        ],
        out_specs=pl.BlockSpec((_EB, _TTB), lambda i, r: (OBS_DIM // _EB + r, i)),
        out_shape=jax.ShapeDtypeStruct((OUT_DIM, BATCH), jnp.float32),
        input_output_aliases={0: 0},
    )(out_t0, embt)
    return out_t


def kernel(obs, task, task_emb_weight):
    table_t = _renorm_table_t(task_emb_weight)
    embt = _sc_gather_t(task, table_t)
    return _assemble(obs, embt).T


# speedup vs baseline: 1.0126x; 1.0126x over previous
import functools

import jax
import jax.numpy as jnp
from jax import lax
from jax.experimental import pallas as pl
from jax.experimental.pallas import tpu as pltpu
from jax.experimental.pallas import tpu_sc as plsc

N_TASKS = 80
TASK_DIM = 96
BATCH = 16384
OBS_DIM = 512
OUT_DIM = OBS_DIM + TASK_DIM
_TPAD = 96
_L = 16


def _renorm_body(w_ref, out_ref):
    w = w_ref[...]
    ss = jnp.sum(w * w, axis=1, keepdims=True)
    scale = jnp.where(ss > 1.0, lax.rsqrt(ss), 1.0)
    wt = (w * scale).T
    out_ref[...] = jnp.concatenate(
        [wt, jnp.zeros((TASK_DIM, _TPAD - N_TASKS), jnp.float32)], axis=1
    )


def _renorm_table_t(w):
    return pl.pallas_call(
        _renorm_body,
        out_shape=jax.ShapeDtypeStruct((TASK_DIM, _TPAD), jnp.float32),
    )(w)


_info = plsc.get_sparse_core_info()
_NC = _info.num_cores
_NS = _info.num_subcores
_NW = _NC * _NS
_B_PER_W = BATCH // _NW


@functools.partial(
    pl.kernel,
    mesh=plsc.VectorSubcoreMesh(core_axis_name="c", subcore_axis_name="s"),
    out_type=jax.ShapeDtypeStruct((TASK_DIM, BATCH), jnp.float32),
    compiler_params=pltpu.CompilerParams(needs_layout_passes=False),
    scratch_types=[
        pltpu.VMEM((_B_PER_W,), jnp.int32),
        pltpu.VMEM((TASK_DIM, _TPAD), jnp.float32),
        pltpu.VMEM((TASK_DIM, _B_PER_W), jnp.float32),
    ],
)
def _sc_gather_t(task_hbm, table_hbm, embt_hbm, idx_v, table_v, embt_v):
    wid = lax.axis_index("s") * _NC + lax.axis_index("c")
    base = wid * _B_PER_W
    pltpu.sync_copy(task_hbm.at[pl.ds(base, _B_PER_W)], idx_v)
    pltpu.sync_copy(table_hbm, table_v)

    def group(g, carry):
        idx16 = idx_v[pl.ds(g * _L, _L)]
        for c in range(TASK_DIM):
            row = jnp.full((_L,), c, jnp.int32)
            embt_v[c, pl.ds(g * _L, _L)] = plsc.load_gather(table_v, [row, idx16])
        return carry

    lax.fori_loop(0, _B_PER_W // _L, group, 0)
    pltpu.sync_copy(embt_v, embt_hbm.at[:, pl.ds(base, _B_PER_W)])


_TB = 4096
_EB = 32
_TTB = 16384


def _obs_body(obs_ref, out_ref):
    out_ref[...] = obs_ref[...].T


def _tail_body(_, embt_ref, out_ref):
    out_ref[...] = embt_ref[...]


def _assemble(obs, embt):
    out_t0 = pl.pallas_call(
        _obs_body,
        grid=(BATCH // _TB,),
        in_specs=[pl.BlockSpec((_TB, OBS_DIM), lambda i: (i, 0))],
        out_specs=pl.BlockSpec((OBS_DIM, _TB), lambda i: (0, i)),
        out_shape=jax.ShapeDtypeStruct((OUT_DIM, BATCH), jnp.float32),
    )(obs)
    out_t = pl.pallas_call(
        _tail_body,
        grid=(BATCH // _TTB, TASK_DIM // _EB),
        in_specs=[
            pl.BlockSpec(memory_space=pl.ANY),
            pl.BlockSpec((_EB, _TTB), lambda i, r: (r, i)),
        ],
        out_specs=pl.BlockSpec((_EB, _TTB), lambda i, r: (OBS_DIM // _EB + r, i)),
        out_shape=jax.ShapeDtypeStruct((OUT_DIM, BATCH), jnp.float32),
        input_output_aliases={0: 0},
    )(out_t0, embt)
    return out_t


def kernel(obs, task, task_emb_weight):
    table_t = _renorm_table_t(task_emb_weight)
    embt = _sc_gather_t(task, table_t)
    return _assemble(obs, embt).T
